# baseline (speedup 1.0000x reference)
import jax
import jax.numpy as jnp
from jax.experimental import pallas as pl
from jax.experimental.pallas import tpu as pltpu

_HEAD = 8
_CHUNK = 512


def _dma_copy_kernel(emb_ref, anchor_ref, out_ref, stage, anchor_v,
                     bulk_sem, head_sem):
    B, S, D = out_ref.shape
    bulk = []
    for b in range(B):
        lo = _HEAD
        while lo < S:
            n = min(_CHUNK, S - lo)
            cp = pltpu.make_async_copy(
                emb_ref.at[b, pl.ds(lo, n), :],
                out_ref.at[b, pl.ds(lo, n), :],
                bulk_sem,
            )
            cp.start()
            bulk.append(cp)
            lo += n
    heads_in = []
    for b in range(B):
        cp = pltpu.make_async_copy(
            emb_ref.at[b, pl.ds(0, _HEAD), :], stage.at[b], head_sem
        )
        cp.start()
        heads_in.append(cp)
    acp = pltpu.make_async_copy(anchor_ref, anchor_v, head_sem)
    acp.start()
    acp.wait()
    for cp in heads_in:
        cp.wait()
    for b in range(B):
        stage[b, 0:1, :] = anchor_v[:, :]
    heads_out = []
    for b in range(B):
        cp = pltpu.make_async_copy(
            stage.at[b], out_ref.at[b, pl.ds(0, _HEAD), :], head_sem
        )
        cp.start()
        heads_out.append(cp)
    for cp in heads_out:
        cp.wait()
    for cp in bulk:
        cp.wait()


def kernel(token_embeddings, style_anchor):
    B, S, D = token_embeddings.shape
    return pl.pallas_call(
        _dma_copy_kernel,
        out_shape=jax.ShapeDtypeStruct(
            token_embeddings.shape, token_embeddings.dtype
        ),
        in_specs=[
            pl.BlockSpec(memory_space=pl.ANY),
            pl.BlockSpec(memory_space=pl.ANY),
        ],
        out_specs=pl.BlockSpec(memory_space=pl.ANY),
        scratch_shapes=[
            pltpu.VMEM((B, _HEAD, D), token_embeddings.dtype),
            pltpu.VMEM((1, D), token_embeddings.dtype),
            pltpu.SemaphoreType.DMA,
            pltpu.SemaphoreType.DMA,
        ],
    )(token_embeddings, style_anchor)


# speedup vs baseline: 43.0875x; 43.0875x over previous
import jax
import jax.numpy as jnp
from jax.experimental import pallas as pl
from jax.experimental.pallas import tpu as pltpu

_BS = 512


def _copy_body(emb_ref, anchor_ref, out_ref):
    j = pl.program_id(1)
    out_ref[...] = emb_ref[...]

    @pl.when(j == 0)
    def _():
        out_ref[0, 0:1, :] = anchor_ref[...]


def kernel(token_embeddings, style_anchor):
    B, S, D = token_embeddings.shape
    grid = (B, S // _BS)
    return pl.pallas_call(
        _copy_body,
        grid=grid,
        in_specs=[
            pl.BlockSpec((1, _BS, D), lambda b, j: (b, j, 0)),
            pl.BlockSpec((1, D), lambda b, j: (0, 0)),
        ],
        out_specs=pl.BlockSpec((1, _BS, D), lambda b, j: (b, j, 0)),
        out_shape=jax.ShapeDtypeStruct(
            token_embeddings.shape, token_embeddings.dtype
        ),
    )(token_embeddings, style_anchor)


# speedup vs baseline: 47.1741x; 1.0948x over previous
import jax
import jax.numpy as jnp
from jax.experimental import pallas as pl
from jax.experimental.pallas import tpu as pltpu

_BS = 1024


def _make_body(blocks_per_batch):
    def _copy_body(emb_ref, anchor_ref, out_ref):
        j = pl.program_id(0)
        out_ref[...] = emb_ref[...]

        @pl.when(j % blocks_per_batch == 0)
        def _():
            out_ref[0:1, :] = anchor_ref[...]

    return _copy_body


def kernel(token_embeddings, style_anchor):
    B, S, D = token_embeddings.shape
    flat = token_embeddings.reshape(B * S, D)
    out = pl.pallas_call(
        _make_body(S // _BS),
        grid=(B * S // _BS,),
        in_specs=[
            pl.BlockSpec((_BS, D), lambda j: (j, 0)),
            pl.BlockSpec((1, D), lambda j: (0, 0)),
        ],
        out_specs=pl.BlockSpec((_BS, D), lambda j: (j, 0)),
        out_shape=jax.ShapeDtypeStruct(flat.shape, flat.dtype),
    )(flat, style_anchor)
    return out.reshape(B, S, D)


# speedup vs baseline: 49.0514x; 1.0398x over previous
import jax
import jax.numpy as jnp
from jax.experimental import pallas as pl
from jax.experimental.pallas import tpu as pltpu

_BS = 2048


def _make_body(blocks_per_batch):
    def _copy_body(emb_ref, anchor_ref, out_ref):
        j = pl.program_id(0)
        out_ref[...] = emb_ref[...]

        @pl.when(j % blocks_per_batch == 0)
        def _():
            out_ref[0:1, :] = anchor_ref[...]

    return _copy_body


def kernel(token_embeddings, style_anchor):
    B, S, D = token_embeddings.shape
    flat = token_embeddings.reshape(B * S, D)
    out = pl.pallas_call(
        _make_body(S // _BS),
        grid=(B * S // _BS,),
        in_specs=[
            pl.BlockSpec((_BS, D), lambda j: (j, 0)),
            pl.BlockSpec((1, D), lambda j: (0, 0)),
        ],
        out_specs=pl.BlockSpec((_BS, D), lambda j: (j, 0)),
        out_shape=jax.ShapeDtypeStruct(flat.shape, flat.dtype),
    )(flat, style_anchor)
    return out.reshape(B, S, D)


# speedup vs baseline: 49.0707x; 1.0004x over previous
import jax
import jax.numpy as jnp
from jax.experimental import pallas as pl
from jax.experimental.pallas import tpu as pltpu

_CHUNK = 2048
_RING = 6


def _make_body(total_rows, rows_per_batch):
    n = total_rows // _CHUNK
    anchor_every = rows_per_batch // _CHUNK

    def _body(emb_ref, anchor_ref, out_ref, bufs, anchor_v,
              load_sems, store_sems, anchor_sem):
        acp = pltpu.make_async_copy(anchor_ref, anchor_v, anchor_sem)
        acp.start()
        loads = {}
        stores = {}
        for i in range(min(_RING, n)):
            loads[i] = pltpu.make_async_copy(
                emb_ref.at[pl.ds(i * _CHUNK, _CHUNK), :],
                bufs.at[i % _RING],
                load_sems.at[i % _RING],
            )
            loads[i].start()
        acp.wait()
        for i in range(n):
            loads[i].wait()
            if i % anchor_every == 0:
                bufs[i % _RING, 0:1, :] = anchor_v[...]
            stores[i] = pltpu.make_async_copy(
                bufs.at[i % _RING],
                out_ref.at[pl.ds(i * _CHUNK, _CHUNK), :],
                store_sems.at[i % _RING],
            )
            stores[i].start()
            nxt = i + _RING
            if nxt < n:
                stores[i].wait()
                loads[nxt] = pltpu.make_async_copy(
                    emb_ref.at[pl.ds(nxt * _CHUNK, _CHUNK), :],
                    bufs.at[nxt % _RING],
                    load_sems.at[nxt % _RING],
                )
                loads[nxt].start()
        for i in range(max(0, n - _RING), n):
            stores[i].wait()

    return _body


def kernel(token_embeddings, style_anchor):
    B, S, D = token_embeddings.shape
    flat = token_embeddings.reshape(B * S, D)
    out = pl.pallas_call(
        _make_body(B * S, S),
        in_specs=[
            pl.BlockSpec(memory_space=pl.ANY),
            pl.BlockSpec(memory_space=pl.ANY),
        ],
        out_specs=pl.BlockSpec(memory_space=pl.ANY),
        out_shape=jax.ShapeDtypeStruct(flat.shape, flat.dtype),
        scratch_shapes=[
            pltpu.VMEM((_RING, _CHUNK, D), flat.dtype),
            pltpu.VMEM((1, D), flat.dtype),
            pltpu.SemaphoreType.DMA((_RING,)),
            pltpu.SemaphoreType.DMA((_RING,)),
            pltpu.SemaphoreType.DMA,
        ],
    )(flat, style_anchor)
    return out.reshape(B, S, D)


# speedup vs baseline: 49.8810x; 1.0165x over previous
import jax
import jax.numpy as jnp
from jax.experimental import pallas as pl
from jax.experimental.pallas import tpu as pltpu

_CHUNK = 1024
_RING = 12


def _make_body(total_rows, rows_per_batch):
    n = total_rows // _CHUNK
    anchor_every = rows_per_batch // _CHUNK

    def _body(emb_ref, anchor_ref, out_ref, bufs, anchor_v,
              load_sems, store_sems, anchor_sem):
        acp = pltpu.make_async_copy(anchor_ref, anchor_v, anchor_sem)
        acp.start()
        loads = {}
        stores = {}
        for i in range(min(_RING, n)):
            loads[i] = pltpu.make_async_copy(
                emb_ref.at[pl.ds(i * _CHUNK, _CHUNK), :],
                bufs.at[i % _RING],
                load_sems.at[i % _RING],
            )
            loads[i].start()
        acp.wait()
        for i in range(n):
            loads[i].wait()
            if i % anchor_every == 0:
                bufs[i % _RING, 0:1, :] = anchor_v[...]
            stores[i] = pltpu.make_async_copy(
                bufs.at[i % _RING],
                out_ref.at[pl.ds(i * _CHUNK, _CHUNK), :],
                store_sems.at[i % _RING],
            )
            stores[i].start()
            nxt = i + _RING
            if nxt < n:
                stores[i].wait()
                loads[nxt] = pltpu.make_async_copy(
                    emb_ref.at[pl.ds(nxt * _CHUNK, _CHUNK), :],
                    bufs.at[nxt % _RING],
                    load_sems.at[nxt % _RING],
                )
                loads[nxt].start()
        for i in range(max(0, n - _RING), n):
            stores[i].wait()

    return _body


def kernel(token_embeddings, style_anchor):
    B, S, D = token_embeddings.shape
    flat = token_embeddings.reshape(B * S, D)
    out = pl.pallas_call(
        _make_body(B * S, S),
        in_specs=[
            pl.BlockSpec(memory_space=pl.ANY),
            pl.BlockSpec(memory_space=pl.ANY),
        ],
        out_specs=pl.BlockSpec(memory_space=pl.ANY),
        out_shape=jax.ShapeDtypeStruct(flat.shape, flat.dtype),
        scratch_shapes=[
            pltpu.VMEM((_RING, _CHUNK, D), flat.dtype),
            pltpu.VMEM((1, D), flat.dtype),
            pltpu.SemaphoreType.DMA((_RING,)),
            pltpu.SemaphoreType.DMA((_RING,)),
            pltpu.SemaphoreType.DMA,
        ],
    )(flat, style_anchor)
    return out.reshape(B, S, D)
